# baseline (speedup 1.0000x reference)
import functools

import jax
import jax.numpy as jnp
from jax import lax
from jax.experimental import pallas as pl
from jax.experimental.pallas import tpu as pltpu
from jax.experimental.pallas import tpu_sc as plsc

N_RULES = 35
D = 512
B = 16
L = 4096

NC = 2
NS = 16
NW = NC * NS

L_CHUNK = L // NW
ROWS = L_CHUNK * B
BLK = 64
LPB = BLK // B
NBLK = ROWS // BLK


def _body(idxT_hbm, len_hbm, table_hbm, out_hbm,
          idx_v, len_v, rid_v, buf, gsem, ssem):
    wid = lax.axis_index("s") * NC + lax.axis_index("c")
    l0 = wid * L_CHUNK
    row0 = wid * ROWS

    pltpu.sync_copy(idxT_hbm.at[pl.ds(l0, L_CHUNK)], idx_v)
    pltpu.sync_copy(len_hbm, len_v)
    lens = len_v[...]

    def blk_body(k, carry):
        rid_row = rid_v.at[k]
        for j in range(LPB):
            lg = jnp.full((B,), l0 + k * LPB + j, jnp.int32)
            row = idx_v[pl.ds(k * LPB + j, 1), :].reshape((B,))
            sel = jnp.where(lg < lens, row,
                            jnp.full((B,), N_RULES, jnp.int32))
            rid_row[pl.ds(j * B, B)] = sel
        pltpu.async_copy(table_hbm.at[rid_row], buf, gsem).wait()
        pltpu.async_copy(
            buf, out_hbm.at[pl.ds(row0 + k * BLK, BLK)], ssem
        ).wait()
        return carry

    lax.fori_loop(0, NBLK, blk_body, 0)


@jax.jit
def kernel(indices, lengths, table):
    idxT = indices.T
    tablez = jnp.concatenate(
        [table, jnp.zeros((1, D), table.dtype)], axis=0)

    mesh = plsc.VectorSubcoreMesh(core_axis_name="c", subcore_axis_name="s")
    out = pl.kernel(
        _body,
        out_type=jax.ShapeDtypeStruct((L * B, D), jnp.float32),
        mesh=mesh,
        scratch_types=[
            pltpu.VMEM((L_CHUNK, B), jnp.int32),
            pltpu.VMEM((B,), jnp.int32),
            pltpu.VMEM((NBLK, BLK), jnp.int32),
            pltpu.VMEM((BLK, D), jnp.float32),
            pltpu.SemaphoreType.DMA,
            pltpu.SemaphoreType.DMA,
        ],
    )(idxT, lengths, tablez)
    return out.reshape(L, B, D)
